# baseline (speedup 1.0000x reference)
import functools

import jax
import jax.numpy as jnp
from jax import lax
from jax.experimental import pallas as pl
from jax.experimental.pallas import tpu as pltpu
from jax.experimental.pallas import tpu_sc as plsc

N = 1024
E = 65536
SDIM = 256
VDIM = 64
EDIM = 16
NA = 16
NB = 5
G = 32

NN2 = N * N + 128
DUMP = N * N
ZSPAN = NN2 // 16
NTILE = 16
EPC = E // 2
EPT = EPC // NTILE
EPT_TAB = E // NTILE
CHK = 128


def _dense_body(s_ref, v3_ref, p_ref, b2_ref, wsh_ref, bsh_ref, wct_ref,
                wat_ref, bat_ref, wbd_ref, bbd_ref, wb0_ref, bb0_ref,
                cpc_ref, hw_ref, at_ref, wcomb_ref, cb_ref):
    f32 = jnp.float32
    h = jax.nn.silu(jnp.dot(s_ref[...], wsh_ref[...], preferred_element_type=f32)
                    + bsh_ref[...])
    w0 = wb0_ref[0:SDIM, :]
    hw_ref[...] = jnp.dot(h, w0, preferred_element_type=f32)
    at_ref[...] = jnp.dot(h, wat_ref[...], preferred_element_type=f32) + bat_ref[...]
    t = v3_ref[...] * wct_ref[...]
    r192 = lax.broadcasted_iota(jnp.int32, (3 * VDIM, 16), 0)
    c192 = lax.broadcasted_iota(jnp.int32, (3 * VDIM, 16), 1)
    sel = (r192 // VDIM == c192).astype(f32)
    cp = jnp.dot(t, sel, preferred_element_type=f32)
    r3 = lax.broadcasted_iota(jnp.int32, (3, 16), 0)
    c3 = lax.broadcasted_iota(jnp.int32, (3, 16), 1)
    sel3 = (r3 == c3).astype(f32)
    cp = cp + jnp.dot(p_ref[...], sel3, preferred_element_type=f32)
    gi = lax.broadcasted_iota(jnp.int32, (N, G), 1)
    oh = (b2_ref[...] == gi).astype(f32)
    dn = (((0,), (0,)), ((), ()))
    cnt = lax.dot_general(oh, jnp.ones((N, 1), f32), dn, preferred_element_type=f32)
    sums = lax.dot_general(oh, cp, dn, preferred_element_type=f32)
    mean = sums / jnp.maximum(cnt, 1.0)
    cpc_ref[...] = cp - jnp.dot(oh, mean, preferred_element_type=f32)
    wbw = jnp.dot(wbd_ref[...], w0, preferred_element_type=f32)
    wdrow = wb0_ref[SDIM:SDIM + 1, :]
    wcomb_ref[...] = jnp.concatenate(
        [wbw, wdrow, jnp.zeros((15, SDIM), f32)], axis=0)
    cb_ref[...] = jnp.dot(bbd_ref[...], w0, preferred_element_type=f32) + bb0_ref[...]


def _bond_body(fp_ref, ef_ref, wcomb_ref, cb_ref, wb1_ref, bb1_ref, out_ref):
    f32 = jnp.float32
    z = (fp_ref[...]
         + jnp.dot(ef_ref[...], wcomb_ref[...], preferred_element_type=f32)
         + cb_ref[...])
    y = jax.nn.silu(z)
    out_ref[...] = jnp.dot(y, wb1_ref[...], preferred_element_type=f32) + bb1_ref[...]


def _edge_body(j_hbm, i_hbm, ep_hbm, hw_hbm, cp_hbm,
               fp_hbm, ef_hbm, tab_hbm,
               jv, iv, buf_a, buf_b, buf_c, buf_d, dbuf, cp_loc,
               rows_i, rows_j, dirb, revb, efb, sem):
    i32 = jnp.int32
    c = lax.axis_index("c")
    sid = lax.axis_index("s")
    coff = c * NN2
    iota16 = lax.iota(i32, 16)
    zero16 = jnp.zeros((16,), i32)
    one16 = zero16 + 1
    two16 = zero16 + 2

    def _zb(t, carry):
        jv[pl.ds(t * 16, 16)] = zero16
        return carry
    lax.fori_loop(0, 256, _zb, None)
    zbase = coff + sid * ZSPAN
    zstarts = [k * 4096 for k in range(16)] + [ZSPAN - 4096]
    hs = [pltpu.async_copy(jv, tab_hbm.at[pl.ds(zbase + st, 4096)], sem)
          for st in zstarts]
    for h in hs:
        h.wait()
    plsc.subcore_barrier()

    eb = sid * EPT_TAB
    pltpu.sync_copy(j_hbm.at[pl.ds(eb, EPT_TAB)], jv)
    pltpu.sync_copy(i_hbm.at[pl.ds(eb, EPT_TAB)], iv)

    def _fill1(t, carry):
        m = t // 8
        l = t - m * 8
        sl = pl.ds(t * 16, 16)
        ll = pl.ds(l * 16, 16)
        buf_a[m, ll] = jv[sl] * N + iv[sl] + coff
        buf_b[m, ll] = iota16 + (eb + t * 16 + 1)
        return carry
    lax.fori_loop(0, 256, _fill1, None)
    for g in range(4):
        hs = [pltpu.async_copy(buf_b.at[m], tab_hbm.at[buf_a.at[m]], sem)
              for m in range(g * 8, g * 8 + 8)]
        for h in hs:
            h.wait()

    lb = c * EPC + sid * EPT
    pltpu.sync_copy(j_hbm.at[pl.ds(lb, EPT)], jv.at[pl.ds(0, EPT)])
    pltpu.sync_copy(i_hbm.at[pl.ds(lb, EPT)], iv.at[pl.ds(0, EPT)])
    pltpu.sync_copy(cp_hbm, cp_loc)

    def _fill2(t, carry):
        m = t // 8
        l = t - m * 8
        sl = pl.ds(t * 16, 16)
        ll = pl.ds(l * 16, 16)
        buf_d[m, ll] = iv[sl]
        buf_d[m + 16, ll] = jv[sl]
        return carry
    lax.fori_loop(0, 128, _fill2, None)

    def _dloop(t, carry):
        sl = pl.ds(t * 16, 16)
        ivv = iv[sl] * 16
        jvv = jv[sl] * 16
        xi = plsc.load_gather(cp_loc, [ivv])
        yi = plsc.load_gather(cp_loc, [ivv + one16])
        zi = plsc.load_gather(cp_loc, [ivv + two16])
        xj = plsc.load_gather(cp_loc, [jvv])
        yj = plsc.load_gather(cp_loc, [jvv + one16])
        zj = plsc.load_gather(cp_loc, [jvv + two16])
        dx = xi - xj
        dy = yi - yj
        dz = zi - zj
        dbuf[sl] = dx * dx + dy * dy + dz * dz
        return carry
    lax.fori_loop(0, 128, _dloop, None)

    def _fchunk(ch, carry):
        base = lb + ch * CHK
        h1 = pltpu.async_copy(hw_hbm.at[buf_d.at[ch]], rows_i, sem)
        h2 = pltpu.async_copy(hw_hbm.at[buf_d.at[ch + 16]], rows_j, sem)
        h1.wait()
        h2.wait()

        def _addl(r, cy):
            for k in range(16):
                sl = pl.ds(k * 16, 16)
                rows_i[r, sl] = rows_i[r, sl] + rows_j[r, sl]
            return cy
        lax.fori_loop(0, CHK, _addl, None)
        pltpu.sync_copy(rows_i, fp_hbm.at[pl.ds(base, CHK)])
        return carry
    lax.fori_loop(0, 16, _fchunk, None)

    plsc.subcore_barrier()
    for _ in range(2):
        for g in range(4):
            hs = [pltpu.async_copy(tab_hbm.at[buf_a.at[m]], buf_c.at[m], sem)
                  for m in range(g * 8, g * 8 + 8)]
            for h in hs:
                h.wait()

        def _fix(t, carry):
            m = t // 8
            ll = pl.ds((t - m * 8) * 16, 16)
            wv = buf_c[m, ll]
            buf_c[m, ll] = jnp.where(buf_b[m, ll] > wv, buf_a[m, ll], coff + DUMP)
            return carry
        lax.fori_loop(0, 256, _fix, None)
        for g in range(4):
            hs = [pltpu.async_copy(buf_b.at[m], tab_hbm.at[buf_c.at[m]], sem)
                  for m in range(g * 8, g * 8 + 8)]
            for h in hs:
                h.wait()
        plsc.subcore_barrier()

    def _fillk(t, carry):
        m = t // 8
        sl = pl.ds(t * 16, 16)
        ll = pl.ds((t - m * 8) * 16, 16)
        buf_a[m, ll] = jv[sl] * N + iv[sl] + coff
        buf_a[m + 16, ll] = iv[sl] * N + jv[sl] + coff
        return carry
    lax.fori_loop(0, 128, _fillk, None)
    for g in range(4):
        hs = [pltpu.async_copy(tab_hbm.at[buf_a.at[m]], buf_b.at[m], sem)
              for m in range(g * 8, g * 8 + 8)]
        for h in hs:
            h.wait()

    def _fill3(t, carry):
        m = t // 8
        ll = pl.ds((t - m * 8) * 16, 16)
        buf_c[m, ll] = buf_b[m, ll] - 1
        wrv = buf_b[m + 16, ll]
        buf_c[m + 16, ll] = jnp.where(wrv == 0, E, wrv - 1)
        return carry
    lax.fori_loop(0, 128, _fill3, None)

    def _echunk(ch, carry):
        base = lb + ch * CHK
        h3 = pltpu.async_copy(ep_hbm.at[buf_c.at[ch]], dirb, sem)
        h4 = pltpu.async_copy(ep_hbm.at[buf_c.at[ch + 16]], revb, sem)
        h3.wait()
        h4.wait()

        def _pere(r, cy):
            efb[r, pl.ds(0, 16)] = (dirb[r, pl.ds(0, 16)]
                                    + revb[r, pl.ds(0, 16)]) * 0.5
            dv = plsc.load_gather(dbuf, [zero16 + (ch * CHK + r)])
            efb[r, pl.ds(16, 16)] = dv
            return cy
        lax.fori_loop(0, CHK, _pere, None)
        pltpu.sync_copy(efb, ef_hbm.at[pl.ds(base, CHK)])
        return carry
    lax.fori_loop(0, 16, _echunk, None)


def _make_edge_kernel():
    return functools.partial(
        pl.kernel,
        out_type=[
            jax.ShapeDtypeStruct((E, SDIM), jnp.float32),
            jax.ShapeDtypeStruct((E, 32), jnp.float32),
            jax.ShapeDtypeStruct((2 * NN2,), jnp.int32),
        ],
        mesh=plsc.VectorSubcoreMesh(core_axis_name="c", subcore_axis_name="s"),
        compiler_params=pltpu.CompilerParams(
            needs_layout_passes=False, use_tc_tiling_on_sc=False),
        scratch_types=[
            pltpu.VMEM((EPT_TAB,), jnp.int32),
            pltpu.VMEM((EPT_TAB,), jnp.int32),
            pltpu.VMEM((32, 128), jnp.int32),
            pltpu.VMEM((32, 128), jnp.int32),
            pltpu.VMEM((32, 128), jnp.int32),
            pltpu.VMEM((32, 128), jnp.int32),
            pltpu.VMEM((EPT,), jnp.float32),
            pltpu.VMEM((N * 16,), jnp.float32),
            pltpu.VMEM((CHK, SDIM), jnp.float32),
            pltpu.VMEM((CHK, SDIM), jnp.float32),
            pltpu.VMEM((CHK, EDIM), jnp.float32),
            pltpu.VMEM((CHK, EDIM), jnp.float32),
            pltpu.VMEM((CHK, 32), jnp.float32),
            pltpu.SemaphoreType.DMA,
        ],
    )(_edge_body)


def kernel(s, v, p, e, batch, edge_index, W_shared, b_shared, W_coords,
           W_atoms, b_atoms, W_bond, b_bond, W_b0, b_b0, W_b1, b_b1):
    f32 = jnp.float32
    b2 = batch.astype(jnp.int32).reshape(N, 1)
    j32 = edge_index[0].astype(jnp.int32)
    i32_ = edge_index[1].astype(jnp.int32)
    v3 = v.reshape(N, 3 * VDIM)
    wct = jnp.tile(W_coords.reshape(-1), 3).reshape(1, 3 * VDIM)
    e_pad = jnp.concatenate([e, jnp.zeros((16, EDIM), f32)], axis=0)

    cpc, hw, atoms, wcomb, cb = pl.pallas_call(
        _dense_body,
        out_shape=[
            jax.ShapeDtypeStruct((N, 16), f32),
            jax.ShapeDtypeStruct((N, SDIM), f32),
            jax.ShapeDtypeStruct((N, NA), f32),
            jax.ShapeDtypeStruct((32, SDIM), f32),
            jax.ShapeDtypeStruct((1, SDIM), f32),
        ],
    )(s, v3, p, b2, W_shared, b_shared.reshape(1, -1), wct,
      W_atoms, b_atoms.reshape(1, -1), W_bond, b_bond.reshape(1, -1),
      W_b0, b_b0.reshape(1, -1))

    fp, ef, _tab = _make_edge_kernel()(j32, i32_, e_pad, hw, cpc.reshape(-1))

    ts = 2048
    bonds = pl.pallas_call(
        _bond_body,
        grid=(E // ts,),
        in_specs=[
            pl.BlockSpec((ts, SDIM), lambda i: (i, 0)),
            pl.BlockSpec((ts, 32), lambda i: (i, 0)),
            pl.BlockSpec((32, SDIM), lambda i: (0, 0)),
            pl.BlockSpec((1, SDIM), lambda i: (0, 0)),
            pl.BlockSpec((SDIM, NB), lambda i: (0, 0)),
            pl.BlockSpec((1, NB), lambda i: (0, 0)),
        ],
        out_specs=pl.BlockSpec((ts, NB), lambda i: (i, 0)),
        out_shape=jax.ShapeDtypeStruct((E, NB), f32),
    )(fp, ef, wcomb, cb, W_b1, b_b1.reshape(1, -1))

    return (cpc[:, :3], atoms, bonds)


# speedup vs baseline: 15.1884x; 15.1884x over previous
import functools

import jax
import jax.numpy as jnp
from jax import lax
from jax.experimental import pallas as pl
from jax.experimental.pallas import tpu as pltpu
from jax.experimental.pallas import tpu_sc as plsc

N = 1024
E = 65536
SDIM = 256
VDIM = 64
EDIM = 16
NA = 16
NB = 5
G = 32

SPREAD = 16384
NN2 = N * N + SPREAD
DUMP = N * N
ZSPAN = NN2 // 16
EMISS = 4096
NTILE = 16
EPC = E // 2
EPT = EPC // NTILE
EPT_TAB = E // NTILE
CHK = 128


def _dense_body(s_ref, v3_ref, p_ref, b2_ref, wsh_ref, bsh_ref, wct_ref,
                wat_ref, bat_ref, wbd_ref, bbd_ref, wb0_ref, bb0_ref,
                cpc_ref, hw_ref, at_ref, wcomb_ref, cb_ref):
    f32 = jnp.float32
    h = jax.nn.silu(jnp.dot(s_ref[...], wsh_ref[...], preferred_element_type=f32)
                    + bsh_ref[...])
    w0 = wb0_ref[0:SDIM, :]
    hw_ref[...] = jnp.dot(h, w0, preferred_element_type=f32)
    at_ref[...] = jnp.dot(h, wat_ref[...], preferred_element_type=f32) + bat_ref[...]
    t = v3_ref[...] * wct_ref[...]
    r192 = lax.broadcasted_iota(jnp.int32, (3 * VDIM, 16), 0)
    c192 = lax.broadcasted_iota(jnp.int32, (3 * VDIM, 16), 1)
    sel = (r192 // VDIM == c192).astype(f32)
    cp = jnp.dot(t, sel, preferred_element_type=f32)
    r3 = lax.broadcasted_iota(jnp.int32, (3, 16), 0)
    c3 = lax.broadcasted_iota(jnp.int32, (3, 16), 1)
    sel3 = (r3 == c3).astype(f32)
    cp = cp + jnp.dot(p_ref[...], sel3, preferred_element_type=f32)
    gi = lax.broadcasted_iota(jnp.int32, (N, G), 1)
    oh = (b2_ref[...] == gi).astype(f32)
    dn = (((0,), (0,)), ((), ()))
    cnt = lax.dot_general(oh, jnp.ones((N, 1), f32), dn, preferred_element_type=f32)
    sums = lax.dot_general(oh, cp, dn, preferred_element_type=f32)
    mean = sums / jnp.maximum(cnt, 1.0)
    cpc_ref[...] = cp - jnp.dot(oh, mean, preferred_element_type=f32)
    wbw = jnp.dot(wbd_ref[...], w0, preferred_element_type=f32)
    wdrow = wb0_ref[SDIM:SDIM + 1, :]
    wcomb_ref[...] = jnp.concatenate(
        [wbw, wdrow, jnp.zeros((15, SDIM), f32)], axis=0)
    cb_ref[...] = jnp.dot(bbd_ref[...], w0, preferred_element_type=f32) + bb0_ref[...]


def _bond_body(fp_ref, ef_ref, wcomb_ref, cb_ref, wb1_ref, bb1_ref, out_ref):
    f32 = jnp.float32
    z = (fp_ref[...]
         + jnp.dot(ef_ref[...], wcomb_ref[...], preferred_element_type=f32)
         + cb_ref[...])
    y = jax.nn.silu(z)
    out_ref[...] = jnp.dot(y, wb1_ref[...], preferred_element_type=f32) + bb1_ref[...]


def _edge_body(j_hbm, i_hbm, ep_hbm, hw_hbm, cp_hbm,
               fp_hbm, ef_hbm, tab_hbm,
               jv, iv, buf_a, buf_b, buf_c, buf_d, dbuf, cp_loc,
               rows_i, rows_j, dirb, revb, efb, sem):
    i32 = jnp.int32
    c = lax.axis_index("c")
    sid = lax.axis_index("s")
    coff = c * NN2
    iota16 = lax.iota(i32, 16)
    zero16 = jnp.zeros((16,), i32)
    one16 = zero16 + 1
    two16 = zero16 + 2

    def _zb(t, carry):
        jv[pl.ds(t * 16, 16)] = zero16
        return carry
    lax.fori_loop(0, 256, _zb, None)
    zbase = coff + sid * ZSPAN
    zstarts = [k * 4096 for k in range(ZSPAN // 4096)] + [ZSPAN - 4096]
    hs = [pltpu.async_copy(jv, tab_hbm.at[pl.ds(zbase + st, 4096)], sem)
          for st in zstarts]
    for h in hs:
        h.wait()
    plsc.subcore_barrier()

    eb = sid * EPT_TAB
    pltpu.sync_copy(j_hbm.at[pl.ds(eb, EPT_TAB)], jv)
    pltpu.sync_copy(i_hbm.at[pl.ds(eb, EPT_TAB)], iv)

    def _fill1(t, carry):
        m = t // 8
        l = t - m * 8
        sl = pl.ds(t * 16, 16)
        ll = pl.ds(l * 16, 16)
        buf_a[m, ll] = jv[sl] * N + iv[sl] + coff
        buf_b[m, ll] = iota16 + (eb + t * 16 + 1)
        return carry
    lax.fori_loop(0, 256, _fill1, None)
    for g in range(4):
        hs = [pltpu.async_copy(buf_b.at[m], tab_hbm.at[buf_a.at[m]], sem)
              for m in range(g * 8, g * 8 + 8)]
        for h in hs:
            h.wait()

    lb = c * EPC + sid * EPT
    pltpu.sync_copy(j_hbm.at[pl.ds(lb, EPT)], jv.at[pl.ds(0, EPT)])
    pltpu.sync_copy(i_hbm.at[pl.ds(lb, EPT)], iv.at[pl.ds(0, EPT)])
    pltpu.sync_copy(cp_hbm, cp_loc)

    def _fill2(t, carry):
        m = t // 8
        l = t - m * 8
        sl = pl.ds(t * 16, 16)
        ll = pl.ds(l * 16, 16)
        buf_d[m, ll] = iv[sl]
        buf_d[m + 16, ll] = jv[sl]
        return carry
    lax.fori_loop(0, 128, _fill2, None)

    def _dloop(t, carry):
        sl = pl.ds(t * 16, 16)
        ivv = iv[sl] * 16
        jvv = jv[sl] * 16
        xi = plsc.load_gather(cp_loc, [ivv])
        yi = plsc.load_gather(cp_loc, [ivv + one16])
        zi = plsc.load_gather(cp_loc, [ivv + two16])
        xj = plsc.load_gather(cp_loc, [jvv])
        yj = plsc.load_gather(cp_loc, [jvv + one16])
        zj = plsc.load_gather(cp_loc, [jvv + two16])
        dx = xi - xj
        dy = yi - yj
        dz = zi - zj
        dbuf[sl] = dx * dx + dy * dy + dz * dz
        return carry
    lax.fori_loop(0, 128, _dloop, None)

    def _fchunk(ch, carry):
        base = lb + ch * CHK
        h1 = pltpu.async_copy(hw_hbm.at[buf_d.at[ch]], rows_i, sem)
        h2 = pltpu.async_copy(hw_hbm.at[buf_d.at[ch + 16]], rows_j, sem)
        h1.wait()
        h2.wait()

        def _addl(r, cy):
            for k in range(16):
                sl = pl.ds(k * 16, 16)
                rows_i[r, sl] = rows_i[r, sl] + rows_j[r, sl]
            return cy
        lax.fori_loop(0, CHK, _addl, None)
        pltpu.sync_copy(rows_i, fp_hbm.at[pl.ds(base, CHK)])
        return carry
    lax.fori_loop(0, 16, _fchunk, None)

    plsc.subcore_barrier()
    for _ in range(2):
        for g in range(4):
            hs = [pltpu.async_copy(tab_hbm.at[buf_a.at[m]], buf_c.at[m], sem)
                  for m in range(g * 8, g * 8 + 8)]
            for h in hs:
                h.wait()

        def _fix(t, carry):
            m = t // 8
            ll = pl.ds((t - m * 8) * 16, 16)
            wv = buf_c[m, ll]
            idvv = buf_b[m, ll]
            dump = coff + DUMP + (idvv & (SPREAD - 1))
            buf_c[m, ll] = jnp.where(idvv > wv, buf_a[m, ll], dump)
            return carry
        lax.fori_loop(0, 256, _fix, None)
        for g in range(4):
            hs = [pltpu.async_copy(buf_b.at[m], tab_hbm.at[buf_c.at[m]], sem)
                  for m in range(g * 8, g * 8 + 8)]
            for h in hs:
                h.wait()
        plsc.subcore_barrier()

    def _fillk(t, carry):
        m = t // 8
        sl = pl.ds(t * 16, 16)
        ll = pl.ds((t - m * 8) * 16, 16)
        buf_a[m, ll] = jv[sl] * N + iv[sl] + coff
        buf_a[m + 16, ll] = iv[sl] * N + jv[sl] + coff
        return carry
    lax.fori_loop(0, 128, _fillk, None)
    for g in range(4):
        hs = [pltpu.async_copy(tab_hbm.at[buf_a.at[m]], buf_b.at[m], sem)
              for m in range(g * 8, g * 8 + 8)]
        for h in hs:
            h.wait()

    def _fill3(t, carry):
        m = t // 8
        ll = pl.ds((t - m * 8) * 16, 16)
        buf_c[m, ll] = buf_b[m, ll] - 1
        wrv = buf_b[m + 16, ll]
        miss = E + ((iota16 + (lb + t * 16)) & (EMISS - 1))
        buf_c[m + 16, ll] = jnp.where(wrv == 0, miss, wrv - 1)
        return carry
    lax.fori_loop(0, 128, _fill3, None)

    def _echunk(ch, carry):
        base = lb + ch * CHK
        h3 = pltpu.async_copy(ep_hbm.at[buf_c.at[ch]], dirb, sem)
        h4 = pltpu.async_copy(ep_hbm.at[buf_c.at[ch + 16]], revb, sem)
        h3.wait()
        h4.wait()

        def _pere(r, cy):
            efb[r, pl.ds(0, 16)] = (dirb[r, pl.ds(0, 16)]
                                    + revb[r, pl.ds(0, 16)]) * 0.5
            dv = plsc.load_gather(dbuf, [zero16 + (ch * CHK + r)])
            efb[r, pl.ds(16, 16)] = dv
            return cy
        lax.fori_loop(0, CHK, _pere, None)
        pltpu.sync_copy(efb, ef_hbm.at[pl.ds(base, CHK)])
        return carry
    lax.fori_loop(0, 16, _echunk, None)


def _make_edge_kernel():
    return functools.partial(
        pl.kernel,
        out_type=[
            jax.ShapeDtypeStruct((E, SDIM), jnp.float32),
            jax.ShapeDtypeStruct((E, 32), jnp.float32),
            jax.ShapeDtypeStruct((2 * NN2,), jnp.int32),
        ],
        mesh=plsc.VectorSubcoreMesh(core_axis_name="c", subcore_axis_name="s"),
        compiler_params=pltpu.CompilerParams(
            needs_layout_passes=False, use_tc_tiling_on_sc=False),
        scratch_types=[
            pltpu.VMEM((EPT_TAB,), jnp.int32),
            pltpu.VMEM((EPT_TAB,), jnp.int32),
            pltpu.VMEM((32, 128), jnp.int32),
            pltpu.VMEM((32, 128), jnp.int32),
            pltpu.VMEM((32, 128), jnp.int32),
            pltpu.VMEM((32, 128), jnp.int32),
            pltpu.VMEM((EPT,), jnp.float32),
            pltpu.VMEM((N * 16,), jnp.float32),
            pltpu.VMEM((CHK, SDIM), jnp.float32),
            pltpu.VMEM((CHK, SDIM), jnp.float32),
            pltpu.VMEM((CHK, EDIM), jnp.float32),
            pltpu.VMEM((CHK, EDIM), jnp.float32),
            pltpu.VMEM((CHK, 32), jnp.float32),
            pltpu.SemaphoreType.DMA,
        ],
    )(_edge_body)


def kernel(s, v, p, e, batch, edge_index, W_shared, b_shared, W_coords,
           W_atoms, b_atoms, W_bond, b_bond, W_b0, b_b0, W_b1, b_b1):
    f32 = jnp.float32
    b2 = batch.astype(jnp.int32).reshape(N, 1)
    j32 = edge_index[0].astype(jnp.int32)
    i32_ = edge_index[1].astype(jnp.int32)
    v3 = v.reshape(N, 3 * VDIM)
    wct = jnp.tile(W_coords.reshape(-1), 3).reshape(1, 3 * VDIM)
    e_pad = jnp.concatenate([e, jnp.zeros((EMISS, EDIM), f32)], axis=0)

    cpc, hw, atoms, wcomb, cb = pl.pallas_call(
        _dense_body,
        out_shape=[
            jax.ShapeDtypeStruct((N, 16), f32),
            jax.ShapeDtypeStruct((N, SDIM), f32),
            jax.ShapeDtypeStruct((N, NA), f32),
            jax.ShapeDtypeStruct((32, SDIM), f32),
            jax.ShapeDtypeStruct((1, SDIM), f32),
        ],
    )(s, v3, p, b2, W_shared, b_shared.reshape(1, -1), wct,
      W_atoms, b_atoms.reshape(1, -1), W_bond, b_bond.reshape(1, -1),
      W_b0, b_b0.reshape(1, -1))

    fp, ef, _tab = _make_edge_kernel()(j32, i32_, e_pad, hw, cpc.reshape(-1))

    ts = 2048
    bonds = pl.pallas_call(
        _bond_body,
        grid=(E // ts,),
        in_specs=[
            pl.BlockSpec((ts, SDIM), lambda i: (i, 0)),
            pl.BlockSpec((ts, 32), lambda i: (i, 0)),
            pl.BlockSpec((32, SDIM), lambda i: (0, 0)),
            pl.BlockSpec((1, SDIM), lambda i: (0, 0)),
            pl.BlockSpec((SDIM, NB), lambda i: (0, 0)),
            pl.BlockSpec((1, NB), lambda i: (0, 0)),
        ],
        out_specs=pl.BlockSpec((ts, NB), lambda i: (i, 0)),
        out_shape=jax.ShapeDtypeStruct((E, NB), f32),
    )(fp, ef, wcomb, cb, W_b1, b_b1.reshape(1, -1))

    return (cpc[:, :3], atoms, bonds)
